# baseline (speedup 1.0000x reference)
import functools

import jax
import jax.numpy as jnp
from jax import lax
from jax.experimental import pallas as pl
from jax.experimental.pallas import tpu as pltpu
from jax.experimental.pallas import tpu_sc as plsc

_B, _T, _DIM = 32, 576, 256
_NSYM, _NCON = 1024, 512
_D = _DIM * 2
_N = _B * _T
_R = 256
_RO = 512
_SCALE = 1.25 / (_N * _D)

_DN_T = (((1,), (1,)), ((), ()))


def _stage2_tables_body(sym_ref, con_ref, cn_ref, cit_ref, d2m_ref):
    sym = sym_ref[...]
    sc = lax.dot_general(sym, con_ref[...], _DN_T,
                         preferred_element_type=jnp.float32)
    sn = jnp.sum(sym * sym, axis=1, keepdims=True)
    d2 = (sn + cn_ref[...]) - 2.0 * sc
    m = jnp.min(d2, axis=1, keepdims=True)
    it = lax.broadcasted_iota(jnp.int32, (_NSYM, _NCON), 1)
    cit_ref[...] = jnp.min(jnp.where(d2 == m, it, _NCON), axis=1,
                           keepdims=True)
    d2m_ref[...] = m


def _stage1_body(zr_ref, zi_ref, sym_ref, bn_ref, cit_ref, d2m_ref,
                 si_ref, ci_ref, sd_ref, cf_ref, dg_ref):
    z = jnp.concatenate([zr_ref[...], zi_ref[...]], axis=1)
    zb = lax.dot_general(z, sym_ref[...], _DN_T,
                         preferred_element_type=jnp.float32)
    rn = jnp.sum(z * z, axis=1, keepdims=True)
    d = (rn + bn_ref[...]) - 2.0 * zb
    mn = jnp.min(d, axis=1, keepdims=True)
    it = lax.broadcasted_iota(jnp.int32, (_R, _NSYM), 1)
    si = jnp.min(jnp.where(d == mn, it, _NSYM), axis=1, keepdims=True)
    oh = it == si
    ci = jnp.min(jnp.where(oh, cit_ref[...], _NCON), axis=1, keepdims=True)
    dg = jnp.min(jnp.where(oh, d2m_ref[...], jnp.inf), axis=1, keepdims=True)
    si_ref[...] = si
    ci_ref[...] = ci
    sd_ref[...] = mn
    cf_ref[...] = 1.0 / (1.0 + mn)
    dg_ref[...] = dg


def _onehot_body(si_ref, ci_ref, ohs_ref, ohc_ref):
    it = lax.broadcasted_iota(jnp.int32, (_RO, _NSYM), 1)
    ohs_ref[...] = (it == si_ref[...]).astype(jnp.float32)
    it2 = lax.broadcasted_iota(jnp.int32, (_RO, _NCON), 1)
    ohc_ref[...] = (it2 == ci_ref[...]).astype(jnp.float32)


_stage2_tables = pl.pallas_call(
    _stage2_tables_body,
    out_shape=[
        jax.ShapeDtypeStruct((_NSYM, 1), jnp.int32),
        jax.ShapeDtypeStruct((_NSYM, 1), jnp.float32),
    ],
)

_stage1 = pl.pallas_call(
    _stage1_body,
    grid=(_N // _R,),
    in_specs=[
        pl.BlockSpec((_R, _DIM), lambda i: (i, 0)),
        pl.BlockSpec((_R, _DIM), lambda i: (i, 0)),
        pl.BlockSpec((_NSYM, _D), lambda i: (0, 0)),
        pl.BlockSpec((1, _NSYM), lambda i: (0, 0)),
        pl.BlockSpec((1, _NSYM), lambda i: (0, 0)),
        pl.BlockSpec((1, _NSYM), lambda i: (0, 0)),
    ],
    out_specs=[
        pl.BlockSpec((_R, 1), lambda i: (i, 0)),
        pl.BlockSpec((_R, 1), lambda i: (i, 0)),
        pl.BlockSpec((_R, 1), lambda i: (i, 0)),
        pl.BlockSpec((_R, 1), lambda i: (i, 0)),
        pl.BlockSpec((_R, 1), lambda i: (i, 0)),
    ],
    out_shape=[
        jax.ShapeDtypeStruct((_N, 1), jnp.int32),
        jax.ShapeDtypeStruct((_N, 1), jnp.int32),
        jax.ShapeDtypeStruct((_N, 1), jnp.float32),
        jax.ShapeDtypeStruct((_N, 1), jnp.float32),
        jax.ShapeDtypeStruct((_N, 1), jnp.float32),
    ],
    compiler_params=pltpu.CompilerParams(
        dimension_semantics=("parallel",)),
)

_onehots = pl.pallas_call(
    _onehot_body,
    grid=(_N // _RO,),
    in_specs=[
        pl.BlockSpec((_RO, 1), lambda i: (i, 0)),
        pl.BlockSpec((_RO, 1), lambda i: (i, 0)),
    ],
    out_specs=[
        pl.BlockSpec((_RO, _NSYM), lambda i: (i, 0)),
        pl.BlockSpec((_RO, _NCON), lambda i: (i, 0)),
    ],
    out_shape=[
        jax.ShapeDtypeStruct((_N, _NSYM), jnp.float32),
        jax.ShapeDtypeStruct((_N, _NCON), jnp.float32),
    ],
    compiler_params=pltpu.CompilerParams(
        dimension_semantics=("parallel",)),
)

_NW = 32
_BPW = _N // _NW
_CH = 48
_NBUF = 4


def _sc_gather_body(table_hbm, idx_hbm, out_hbm, idx_v, *bufs_and_sems):
    bufs = bufs_and_sems[:_NBUF]
    gsems = bufs_and_sems[_NBUF:2 * _NBUF]
    ssems = bufs_and_sems[2 * _NBUF:3 * _NBUF]
    wid = lax.axis_index("s") * 2 + lax.axis_index("c")
    base = wid * _BPW
    pltpu.sync_copy(idx_hbm.at[pl.ds(base, _BPW)], idx_v)
    nch = _BPW // _CH
    gcp, scp = {}, {}

    def start_gather(c):
        gcp[c] = pltpu.async_copy(
            table_hbm.at[idx_v.at[pl.ds(c * _CH, _CH)]],
            bufs[c % _NBUF], gsems[c % _NBUF])

    for c in range(min(_NBUF, nch)):
        start_gather(c)
    for c in range(nch):
        gcp[c].wait()
        scp[c] = pltpu.async_copy(
            bufs[c % _NBUF], out_hbm.at[pl.ds(base + c * _CH, _CH)],
            ssems[c % _NBUF])
        if c + _NBUF < nch:
            scp[c].wait()
            start_gather(c + _NBUF)
    for c in range(max(0, nch - _NBUF), nch):
        scp[c].wait()


@functools.cache
def _sc_gather():
    return pl.kernel(
        _sc_gather_body,
        out_type=jax.ShapeDtypeStruct((_N, _D), jnp.float32),
        mesh=plsc.VectorSubcoreMesh(core_axis_name="c", subcore_axis_name="s"),
        scratch_types=(
            [pltpu.VMEM((_BPW,), jnp.int32)]
            + [pltpu.VMEM((_CH, _D), jnp.float32)] * _NBUF
            + [pltpu.SemaphoreType.DMA] * (2 * _NBUF)
        ),
    )


def kernel(z_real, z_imag, sym, con):
    zr = z_real.reshape(_N, _DIM)
    zi = z_imag.reshape(_N, _DIM)
    bn = jnp.sum(sym**2, axis=-1).reshape(1, _NSYM)
    cn = jnp.sum(con**2, axis=-1).reshape(1, _NCON)
    cit, d2m = _stage2_tables(sym, con, cn)
    si2, ci2, sd2, cf2, dg2 = _stage1(
        zr, zi, sym, bn, cit.reshape(1, _NSYM), d2m.reshape(1, _NSYM))
    zs = _sc_gather()(sym, si2.reshape(_N))
    ohs, ohc = _onehots(si2, ci2)
    out_c = lax.complex(zs[:, :_DIM], zs[:, _DIM:]).reshape(_B, _T, _DIM)
    ls = _SCALE * jnp.sum(sd2)
    lc = _SCALE * jnp.sum(dg2)
    return (out_c,
            ohs.reshape(_B, _T, _NSYM),
            ohc.reshape(_B, _T, _NCON),
            ls, lc,
            si2.reshape(_B, _T),
            ci2.reshape(_B, _T),
            cf2.reshape(_B, _T))


# speedup vs baseline: 1.0302x; 1.0302x over previous
import functools

import jax
import jax.numpy as jnp
from jax import lax
from jax.experimental import pallas as pl
from jax.experimental.pallas import tpu as pltpu
from jax.experimental.pallas import tpu_sc as plsc

_B, _T, _DIM = 32, 576, 256
_NSYM, _NCON = 1024, 512
_D = _DIM * 2
_N = _B * _T
_R = 256
_RO = 512
_SCALE = 1.25 / (_N * _D)

_DN_T = (((1,), (1,)), ((), ()))


def _stage2_tables_body(sym_ref, con_ref, cn_ref, pkt_ref, d2m_ref):
    sym = sym_ref[...]
    sc = lax.dot_general(sym, con_ref[...], _DN_T,
                         preferred_element_type=jnp.float32)
    sn = jnp.sum(sym * sym, axis=1, keepdims=True)
    d2 = (sn + cn_ref[...]) - 2.0 * sc
    m = jnp.min(d2, axis=1, keepdims=True)
    it = lax.broadcasted_iota(jnp.int32, (_NSYM, _NCON), 1)
    cit = jnp.min(jnp.where(d2 == m, it, _NCON), axis=1, keepdims=True)
    row = lax.broadcasted_iota(jnp.int32, (_NSYM, 1), 0)
    pkt_ref[...] = row * 1024 + cit
    d2m_ref[...] = m


def _stage1_body(zr_ref, zi_ref, sym_ref, bn_ref, pkt_ref, d2m_ref,
                 si_ref, ci_ref, sd_ref, cf_ref, dg_ref):
    z = jnp.concatenate([zr_ref[...], zi_ref[...]], axis=1)
    zb = lax.dot_general(z, sym_ref[...], _DN_T,
                         preferred_element_type=jnp.float32)
    rn = jnp.sum(z * z, axis=1, keepdims=True)
    d = (rn + bn_ref[...]) - 2.0 * zb
    mn = jnp.min(d, axis=1, keepdims=True)
    pk = jnp.min(jnp.where(d == mn, pkt_ref[...], _NSYM * 1024),
                 axis=1, keepdims=True)
    si_ref[...] = lax.shift_right_logical(pk, 10)
    ci_ref[...] = lax.bitwise_and(pk, 1023)
    sd_ref[...] = mn
    cf_ref[...] = 1.0 / (1.0 + mn)
    dg_ref[...] = jnp.min(jnp.where(d == mn, d2m_ref[...], jnp.inf),
                          axis=1, keepdims=True)


def _onehot_body(si_ref, ci_ref, ohs_ref, ohc_ref):
    it = lax.broadcasted_iota(jnp.int32, (_RO, _NSYM), 1)
    ohs_ref[...] = (it == si_ref[...]).astype(jnp.float32)
    it2 = lax.broadcasted_iota(jnp.int32, (_RO, _NCON), 1)
    ohc_ref[...] = (it2 == ci_ref[...]).astype(jnp.float32)


_stage2_tables = pl.pallas_call(
    _stage2_tables_body,
    out_shape=[
        jax.ShapeDtypeStruct((_NSYM, 1), jnp.int32),
        jax.ShapeDtypeStruct((_NSYM, 1), jnp.float32),
    ],
)

_stage1 = pl.pallas_call(
    _stage1_body,
    grid=(_N // _R,),
    in_specs=[
        pl.BlockSpec((_R, _DIM), lambda i: (i, 0)),
        pl.BlockSpec((_R, _DIM), lambda i: (i, 0)),
        pl.BlockSpec((_NSYM, _D), lambda i: (0, 0)),
        pl.BlockSpec((1, _NSYM), lambda i: (0, 0)),
        pl.BlockSpec((1, _NSYM), lambda i: (0, 0)),
        pl.BlockSpec((1, _NSYM), lambda i: (0, 0)),
    ],
    out_specs=[
        pl.BlockSpec((_R, 1), lambda i: (i, 0)),
        pl.BlockSpec((_R, 1), lambda i: (i, 0)),
        pl.BlockSpec((_R, 1), lambda i: (i, 0)),
        pl.BlockSpec((_R, 1), lambda i: (i, 0)),
        pl.BlockSpec((_R, 1), lambda i: (i, 0)),
    ],
    out_shape=[
        jax.ShapeDtypeStruct((_N, 1), jnp.int32),
        jax.ShapeDtypeStruct((_N, 1), jnp.int32),
        jax.ShapeDtypeStruct((_N, 1), jnp.float32),
        jax.ShapeDtypeStruct((_N, 1), jnp.float32),
        jax.ShapeDtypeStruct((_N, 1), jnp.float32),
    ],
    compiler_params=pltpu.CompilerParams(
        dimension_semantics=("parallel",)),
)

_onehots = pl.pallas_call(
    _onehot_body,
    grid=(_N // _RO,),
    in_specs=[
        pl.BlockSpec((_RO, 1), lambda i: (i, 0)),
        pl.BlockSpec((_RO, 1), lambda i: (i, 0)),
    ],
    out_specs=[
        pl.BlockSpec((_RO, _NSYM), lambda i: (i, 0)),
        pl.BlockSpec((_RO, _NCON), lambda i: (i, 0)),
    ],
    out_shape=[
        jax.ShapeDtypeStruct((_N, _NSYM), jnp.float32),
        jax.ShapeDtypeStruct((_N, _NCON), jnp.float32),
    ],
    compiler_params=pltpu.CompilerParams(
        dimension_semantics=("parallel",)),
)

_NW = 32
_BPW = _N // _NW
_CH = 96
_NBUF = 2
_L = 16


def _sc_gather_body(symr_hbm, symi_hbm, idx_hbm,
                    zre_hbm, zim_hbm,
                    idx_v, br0, bi0, br1, bi1,
                    gs0, gs1, ss0, ss1):
    bufs = ((br0, bi0), (br1, bi1))
    gsems = (gs0, gs1)
    ssems = (ss0, ss1)
    wid = lax.axis_index("s") * 2 + lax.axis_index("c")
    base = wid * _BPW
    pltpu.sync_copy(idx_hbm.at[pl.ds(base, _BPW)], idx_v)
    nch = _BPW // _CH
    gcp, scp = {}, {}

    def start_gather(c):
        s = c % _NBUF
        idx = idx_v.at[pl.ds(c * _CH, _CH)]
        gcp[c] = (
            pltpu.async_copy(symr_hbm.at[idx], bufs[s][0], gsems[s]),
            pltpu.async_copy(symi_hbm.at[idx], bufs[s][1], gsems[s]),
        )

    for c in range(min(_NBUF, nch)):
        start_gather(c)
    for c in range(nch):
        s = c % _NBUF
        gcp[c][0].wait()
        gcp[c][1].wait()
        dst = pl.ds(base + c * _CH, _CH)
        scp[c] = (
            pltpu.async_copy(bufs[s][0], zre_hbm.at[dst], ssems[s]),
            pltpu.async_copy(bufs[s][1], zim_hbm.at[dst], ssems[s]),
        )
        if c + _NBUF < nch:
            scp[c][0].wait()
            scp[c][1].wait()
            start_gather(c + _NBUF)
    for c in range(max(0, nch - _NBUF), nch):
        scp[c][0].wait()
        scp[c][1].wait()


@functools.cache
def _sc_gather():
    return pl.kernel(
        _sc_gather_body,
        out_type=(
            jax.ShapeDtypeStruct((_N, _DIM), jnp.float32),
            jax.ShapeDtypeStruct((_N, _DIM), jnp.float32),
        ),
        mesh=plsc.VectorSubcoreMesh(core_axis_name="c", subcore_axis_name="s"),
        scratch_types=(
            [pltpu.VMEM((_BPW,), jnp.int32)]
            + [pltpu.VMEM((_CH, _DIM), jnp.float32)] * 4
            + [pltpu.SemaphoreType.DMA] * 4
        ),
    )


def kernel(z_real, z_imag, sym, con):
    zr = z_real.reshape(_N, _DIM)
    zi = z_imag.reshape(_N, _DIM)
    bn = jnp.sum(sym**2, axis=-1).reshape(1, _NSYM)
    cn = jnp.sum(con**2, axis=-1).reshape(1, _NCON)
    pkt, d2m = _stage2_tables(sym, con, cn)
    si2, ci2, sd2, cf2, dg = _stage1(
        zr, zi, sym, bn, pkt.reshape(1, _NSYM), d2m.reshape(1, _NSYM))
    zre, zim = _sc_gather()(sym[:, :_DIM], sym[:, _DIM:], si2.reshape(_N))
    ohs, ohc = _onehots(si2, ci2)
    out_c = lax.complex(zre, zim).reshape(_B, _T, _DIM)
    ls = _SCALE * jnp.sum(sd2)
    lc = _SCALE * jnp.sum(dg)
    return (out_c,
            ohs.reshape(_B, _T, _NSYM),
            ohc.reshape(_B, _T, _NCON),
            ls, lc,
            si2.reshape(_B, _T),
            ci2.reshape(_B, _T),
            cf2.reshape(_B, _T))


# speedup vs baseline: 1.0619x; 1.0308x over previous
import functools

import jax
import jax.numpy as jnp
from jax import lax
from jax.experimental import pallas as pl
from jax.experimental.pallas import tpu as pltpu
from jax.experimental.pallas import tpu_sc as plsc

_B, _T, _DIM = 32, 576, 256
_NSYM, _NCON = 1024, 512
_D = _DIM * 2
_N = _B * _T
_R = 256
_RL = _R // 128
_NL = _N // 128
_RO = 512
_SCALE = 1.25 / (_N * _D)

_DN_T = (((1,), (1,)), ((), ()))


def _stage2_tables_body(sym_ref, con_ref, cn_ref, pkt_ref, d2m_ref):
    sym = sym_ref[...]
    sc = lax.dot_general(sym, con_ref[...], _DN_T,
                         preferred_element_type=jnp.float32)
    sn = jnp.sum(sym * sym, axis=1, keepdims=True)
    d2 = (sn + cn_ref[...]) - 2.0 * sc
    m = jnp.min(d2, axis=1, keepdims=True)
    it = lax.broadcasted_iota(jnp.int32, (_NSYM, _NCON), 1)
    cit = jnp.min(jnp.where(d2 == m, it, _NCON), axis=1, keepdims=True)
    row = lax.broadcasted_iota(jnp.int32, (_NSYM, 1), 0)
    pkt_ref[...] = row * 1024 + cit
    d2m_ref[...] = m


def _stage1_body(zr_ref, zi_ref, sym_ref, bn_ref, pkt_ref, d2m_ref,
                 si_ref, ci_ref, sd_ref, cf_ref, dg_ref):
    z = jnp.concatenate([zr_ref[...], zi_ref[...]], axis=1)
    zb = lax.dot_general(z, sym_ref[...], _DN_T,
                         preferred_element_type=jnp.float32)
    rn = jnp.sum(z * z, axis=1, keepdims=True)
    d = (rn + bn_ref[...]) - 2.0 * zb
    mn = jnp.min(d, axis=1, keepdims=True)
    pk = jnp.min(jnp.where(d == mn, pkt_ref[...], _NSYM * 1024),
                 axis=1, keepdims=True)
    dg = jnp.min(jnp.where(d == mn, d2m_ref[...], jnp.inf),
                 axis=1, keepdims=True)
    si_ref[...] = jnp.reshape(lax.shift_right_logical(pk, 10), (1, _RL, 128))
    ci_ref[...] = jnp.reshape(lax.bitwise_and(pk, 1023), (1, _RL, 128))
    sd_ref[...] = jnp.reshape(mn, (1, _RL, 128))
    cf_ref[...] = jnp.reshape(1.0 / (1.0 + mn), (1, _RL, 128))
    dg_ref[...] = jnp.reshape(dg, (1, _RL, 128))


def _onehot_body(si_ref, ci_ref, ohs_ref, ohc_ref):
    it = lax.broadcasted_iota(jnp.int32, (_RO, _NSYM), 1)
    ohs_ref[...] = (it == si_ref[...]).astype(jnp.float32)
    it2 = lax.broadcasted_iota(jnp.int32, (_RO, _NCON), 1)
    ohc_ref[...] = (it2 == ci_ref[...]).astype(jnp.float32)


_stage2_tables = pl.pallas_call(
    _stage2_tables_body,
    out_shape=[
        jax.ShapeDtypeStruct((_NSYM, 1), jnp.int32),
        jax.ShapeDtypeStruct((_NSYM, 1), jnp.float32),
    ],
)

_stage1 = pl.pallas_call(
    _stage1_body,
    grid=(_N // _R,),
    in_specs=[
        pl.BlockSpec((_R, _DIM), lambda i: (i, 0)),
        pl.BlockSpec((_R, _DIM), lambda i: (i, 0)),
        pl.BlockSpec((_NSYM, _D), lambda i: (0, 0)),
        pl.BlockSpec((1, _NSYM), lambda i: (0, 0)),
        pl.BlockSpec((1, _NSYM), lambda i: (0, 0)),
        pl.BlockSpec((1, _NSYM), lambda i: (0, 0)),
    ],
    out_specs=[
        pl.BlockSpec((1, _RL, 128), lambda i: (i, 0, 0)),
        pl.BlockSpec((1, _RL, 128), lambda i: (i, 0, 0)),
        pl.BlockSpec((1, _RL, 128), lambda i: (i, 0, 0)),
        pl.BlockSpec((1, _RL, 128), lambda i: (i, 0, 0)),
        pl.BlockSpec((1, _RL, 128), lambda i: (i, 0, 0)),
    ],
    out_shape=[
        jax.ShapeDtypeStruct((_N // _R, _RL, 128), jnp.int32),
        jax.ShapeDtypeStruct((_N // _R, _RL, 128), jnp.int32),
        jax.ShapeDtypeStruct((_N // _R, _RL, 128), jnp.float32),
        jax.ShapeDtypeStruct((_N // _R, _RL, 128), jnp.float32),
        jax.ShapeDtypeStruct((_N // _R, _RL, 128), jnp.float32),
    ],
    compiler_params=pltpu.CompilerParams(
        dimension_semantics=("parallel",)),
)

_onehots = pl.pallas_call(
    _onehot_body,
    grid=(_N // _RO,),
    in_specs=[
        pl.BlockSpec((_RO, 1), lambda i: (i, 0)),
        pl.BlockSpec((_RO, 1), lambda i: (i, 0)),
    ],
    out_specs=[
        pl.BlockSpec((_RO, _NSYM), lambda i: (i, 0)),
        pl.BlockSpec((_RO, _NCON), lambda i: (i, 0)),
    ],
    out_shape=[
        jax.ShapeDtypeStruct((_N, _NSYM), jnp.float32),
        jax.ShapeDtypeStruct((_N, _NCON), jnp.float32),
    ],
    compiler_params=pltpu.CompilerParams(
        dimension_semantics=("parallel",)),
)

_NW = 32
_BPW = _N // _NW
_CH = 96
_NBUF = 2


def _sc_gather_body(table_hbm, idx_hbm, out_hbm, idx_v, b0, b1,
                    gs0, gs1, ss0, ss1):
    bufs = (b0, b1)
    gsems = (gs0, gs1)
    ssems = (ss0, ss1)
    wid = lax.axis_index("s") * 2 + lax.axis_index("c")
    base = wid * _BPW
    pltpu.sync_copy(idx_hbm.at[pl.ds(base, _BPW)], idx_v)
    nch = _BPW // _CH
    gcp, scp = {}, {}

    def start_gather(c):
        s = c % _NBUF
        gcp[c] = pltpu.async_copy(
            table_hbm.at[idx_v.at[pl.ds(c * _CH, _CH)]], bufs[s], gsems[s])

    for c in range(min(_NBUF, nch)):
        start_gather(c)
    for c in range(nch):
        s = c % _NBUF
        gcp[c].wait()
        scp[c] = pltpu.async_copy(
            bufs[s], out_hbm.at[pl.ds(base + c * _CH, _CH)], ssems[s])
        if c + _NBUF < nch:
            scp[c].wait()
            start_gather(c + _NBUF)
    for c in range(max(0, nch - _NBUF), nch):
        scp[c].wait()


@functools.cache
def _sc_gather():
    return pl.kernel(
        _sc_gather_body,
        out_type=jax.ShapeDtypeStruct((_N, _D), jnp.float32),
        mesh=plsc.VectorSubcoreMesh(core_axis_name="c", subcore_axis_name="s"),
        scratch_types=(
            [pltpu.VMEM((_BPW,), jnp.int32)]
            + [pltpu.VMEM((_CH, _D), jnp.float32)] * _NBUF
            + [pltpu.SemaphoreType.DMA] * (2 * _NBUF)
        ),
    )


def kernel(z_real, z_imag, sym, con):
    zr = z_real.reshape(_N, _DIM)
    zi = z_imag.reshape(_N, _DIM)
    bn = jnp.sum(sym**2, axis=-1).reshape(1, _NSYM)
    cn = jnp.sum(con**2, axis=-1).reshape(1, _NCON)
    pkt, d2m = _stage2_tables(sym, con, cn)
    si2, ci2, sd2, cf2, dg = _stage1(
        zr, zi, sym, bn, pkt.reshape(1, _NSYM), d2m.reshape(1, _NSYM))
    zs = _sc_gather()(sym, si2.reshape(_N))
    ohs, ohc = _onehots(si2.reshape(_N, 1), ci2.reshape(_N, 1))
    out_c = lax.complex(zs[:, :_DIM], zs[:, _DIM:]).reshape(_B, _T, _DIM)
    ls = _SCALE * jnp.sum(sd2)
    lc = _SCALE * jnp.sum(dg)
    return (out_c,
            ohs.reshape(_B, _T, _NSYM),
            ohc.reshape(_B, _T, _NCON),
            ls, lc,
            si2.reshape(_B, _T),
            ci2.reshape(_B, _T),
            cf2.reshape(_B, _T))


# speedup vs baseline: 1.1202x; 1.0549x over previous
import functools

import jax
import jax.numpy as jnp
from jax import lax
from jax.experimental import pallas as pl
from jax.experimental.pallas import tpu as pltpu
from jax.experimental.pallas import tpu_sc as plsc

_B, _T, _DIM = 32, 576, 256
_NSYM, _NCON = 1024, 512
_D = _DIM * 2
_N = _B * _T
_R = 256
_RL = _R // 128
_NL = _N // 128
_RO = 512
_SCALE = 1.25 / (_N * _D)

_DN_T = (((1,), (1,)), ((), ()))


def _stage2_tables_body(sym_ref, con_ref, cn_ref, pkt_ref, d2m_ref):
    sym = sym_ref[...]
    sc = lax.dot_general(sym, con_ref[...], _DN_T,
                         preferred_element_type=jnp.float32)
    sn = jnp.sum(sym * sym, axis=1, keepdims=True)
    d2 = (sn + cn_ref[...]) - 2.0 * sc
    m = jnp.min(d2, axis=1, keepdims=True)
    it = lax.broadcasted_iota(jnp.int32, (_NSYM, _NCON), 1)
    cit = jnp.min(jnp.where(d2 == m, it, _NCON), axis=1, keepdims=True)
    row = lax.broadcasted_iota(jnp.int32, (_NSYM, 1), 0)
    pkt_ref[...] = row * 1024 + cit
    d2m_ref[...] = m


def _stage1_body(zr_ref, zi_ref, sym_ref, bn_ref, pkt_ref, d2m_ref,
                 si_ref, ci_ref, sd_ref, cf_ref, dg_ref):
    z = jnp.concatenate([zr_ref[...], zi_ref[...]], axis=1)
    zb = lax.dot_general(z, sym_ref[...], _DN_T,
                         preferred_element_type=jnp.float32)
    rn = jnp.sum(z * z, axis=1, keepdims=True)
    d = (rn + bn_ref[...]) - 2.0 * zb
    mn = jnp.min(d, axis=1, keepdims=True)
    pk = jnp.min(jnp.where(d == mn, pkt_ref[...], _NSYM * 1024),
                 axis=1, keepdims=True)
    dg = jnp.min(jnp.where(d == mn, d2m_ref[...], jnp.inf),
                 axis=1, keepdims=True)
    si_ref[...] = jnp.reshape(lax.shift_right_logical(pk, 10), (1, _RL, 128))
    ci_ref[...] = jnp.reshape(lax.bitwise_and(pk, 1023), (1, _RL, 128))
    sd_ref[...] = jnp.reshape(mn, (1, _RL, 128))
    cf_ref[...] = jnp.reshape(1.0 / (1.0 + mn), (1, _RL, 128))
    dg_ref[...] = jnp.reshape(dg, (1, _RL, 128))


def _onehot_body(si_ref, ci_ref, ohs_ref, ohc_ref):
    it = lax.broadcasted_iota(jnp.int32, (_RO, _NSYM), 1)
    ohs_ref[...] = (it == si_ref[...]).astype(jnp.float32)
    it2 = lax.broadcasted_iota(jnp.int32, (_RO, _NCON), 1)
    ohc_ref[...] = (it2 == ci_ref[...]).astype(jnp.float32)


_stage2_tables = pl.pallas_call(
    _stage2_tables_body,
    out_shape=[
        jax.ShapeDtypeStruct((_NSYM, 1), jnp.int32),
        jax.ShapeDtypeStruct((_NSYM, 1), jnp.float32),
    ],
)

_stage1 = pl.pallas_call(
    _stage1_body,
    grid=(_N // _R,),
    in_specs=[
        pl.BlockSpec((_R, _DIM), lambda i: (i, 0)),
        pl.BlockSpec((_R, _DIM), lambda i: (i, 0)),
        pl.BlockSpec((_NSYM, _D), lambda i: (0, 0)),
        pl.BlockSpec((1, _NSYM), lambda i: (0, 0)),
        pl.BlockSpec((1, _NSYM), lambda i: (0, 0)),
        pl.BlockSpec((1, _NSYM), lambda i: (0, 0)),
    ],
    out_specs=[
        pl.BlockSpec((1, _RL, 128), lambda i: (i, 0, 0)),
        pl.BlockSpec((1, _RL, 128), lambda i: (i, 0, 0)),
        pl.BlockSpec((1, _RL, 128), lambda i: (i, 0, 0)),
        pl.BlockSpec((1, _RL, 128), lambda i: (i, 0, 0)),
        pl.BlockSpec((1, _RL, 128), lambda i: (i, 0, 0)),
    ],
    out_shape=[
        jax.ShapeDtypeStruct((_N // _R, _RL, 128), jnp.int32),
        jax.ShapeDtypeStruct((_N // _R, _RL, 128), jnp.int32),
        jax.ShapeDtypeStruct((_N // _R, _RL, 128), jnp.float32),
        jax.ShapeDtypeStruct((_N // _R, _RL, 128), jnp.float32),
        jax.ShapeDtypeStruct((_N // _R, _RL, 128), jnp.float32),
    ],
    compiler_params=pltpu.CompilerParams(
        dimension_semantics=("parallel",)),
)

_onehots = pl.pallas_call(
    _onehot_body,
    grid=(_N // _RO,),
    in_specs=[
        pl.BlockSpec((_RO, 1), lambda i: (i, 0)),
        pl.BlockSpec((_RO, 1), lambda i: (i, 0)),
    ],
    out_specs=[
        pl.BlockSpec((_RO, _NSYM), lambda i: (i, 0)),
        pl.BlockSpec((_RO, _NCON), lambda i: (i, 0)),
    ],
    out_shape=[
        jax.ShapeDtypeStruct((_N, _NSYM), jnp.float32),
        jax.ShapeDtypeStruct((_N, _NCON), jnp.float32),
    ],
    compiler_params=pltpu.CompilerParams(
        dimension_semantics=("parallel",)),
)

_NW = 32
_BPW = _N // _NW
_CH = 72
_NBUF = 3


def _sc_gather_body(table_hbm, idx_hbm, zre_hbm, zim_hbm, idx_v, b0, b1, b2,
                    gs0, gs1, gs2, ss0, ss1, ss2):
    bufs = (b0, b1, b2)
    gsems = (gs0, gs1, gs2)
    ssems = (ss0, ss1, ss2)
    wid = lax.axis_index("s") * 2 + lax.axis_index("c")
    base = wid * _BPW
    pltpu.sync_copy(idx_hbm.at[pl.ds(base, _BPW)], idx_v)
    nch = _BPW // _CH
    gcp, scp = {}, {}

    def start_gather(c):
        s = c % _NBUF
        gcp[c] = pltpu.async_copy(
            table_hbm.at[idx_v.at[pl.ds(c * _CH, _CH)]], bufs[s], gsems[s])

    for c in range(min(_NBUF, nch)):
        start_gather(c)
    for c in range(nch):
        s = c % _NBUF
        gcp[c].wait()
        dst = pl.ds(base + c * _CH, _CH)
        scp[c] = (
            pltpu.async_copy(bufs[s].at[:, pl.ds(0, _DIM)],
                             zre_hbm.at[dst], ssems[s]),
            pltpu.async_copy(bufs[s].at[:, pl.ds(_DIM, _DIM)],
                             zim_hbm.at[dst], ssems[s]),
        )
        if c + _NBUF < nch:
            scp[c][0].wait()
            scp[c][1].wait()
            start_gather(c + _NBUF)
    for c in range(max(0, nch - _NBUF), nch):
        scp[c][0].wait()
        scp[c][1].wait()


@functools.cache
def _sc_gather():
    return pl.kernel(
        _sc_gather_body,
        out_type=(
            jax.ShapeDtypeStruct((_N, _DIM), jnp.float32),
            jax.ShapeDtypeStruct((_N, _DIM), jnp.float32),
        ),
        mesh=plsc.VectorSubcoreMesh(core_axis_name="c", subcore_axis_name="s"),
        scratch_types=(
            [pltpu.VMEM((_BPW,), jnp.int32)]
            + [pltpu.VMEM((_CH, _D), jnp.float32)] * _NBUF
            + [pltpu.SemaphoreType.DMA] * (2 * _NBUF)
        ),
    )


def kernel(z_real, z_imag, sym, con):
    zr = z_real.reshape(_N, _DIM)
    zi = z_imag.reshape(_N, _DIM)
    bn = jnp.sum(sym**2, axis=-1).reshape(1, _NSYM)
    cn = jnp.sum(con**2, axis=-1).reshape(1, _NCON)
    pkt, d2m = _stage2_tables(sym, con, cn)
    si2, ci2, sd2, cf2, dg = _stage1(
        zr, zi, sym, bn, pkt.reshape(1, _NSYM), d2m.reshape(1, _NSYM))
    zre, zim = _sc_gather()(sym, si2.reshape(_N))
    ohs, ohc = _onehots(si2.reshape(_N, 1), ci2.reshape(_N, 1))
    out_c = lax.complex(zre, zim).reshape(_B, _T, _DIM)
    ls = _SCALE * jnp.sum(sd2)
    lc = _SCALE * jnp.sum(dg)
    return (out_c,
            ohs.reshape(_B, _T, _NSYM),
            ohc.reshape(_B, _T, _NCON),
            ls, lc,
            si2.reshape(_B, _T),
            ci2.reshape(_B, _T),
            cf2.reshape(_B, _T))


# speedup vs baseline: 1.1216x; 1.0013x over previous
import functools

import jax
import jax.numpy as jnp
from jax import lax
from jax.experimental import pallas as pl
from jax.experimental.pallas import tpu as pltpu
from jax.experimental.pallas import tpu_sc as plsc

_B, _T, _DIM = 32, 576, 256
_NSYM, _NCON = 1024, 512
_D = _DIM * 2
_N = _B * _T
_R = 256
_RL = _R // 128
_NL = _N // 128
_RO = 512
_SCALE = 1.25 / (_N * _D)

_DN_T = (((1,), (1,)), ((), ()))


def _stage2_tables_body(sym_ref, con_ref, cn_ref, pkt_ref, d2m_ref):
    sym = sym_ref[...]
    sc = lax.dot_general(sym, con_ref[...], _DN_T,
                         preferred_element_type=jnp.float32)
    sn = jnp.sum(sym * sym, axis=1, keepdims=True)
    d2 = (sn + cn_ref[...]) - 2.0 * sc
    m = jnp.min(d2, axis=1, keepdims=True)
    it = lax.broadcasted_iota(jnp.int32, (_NSYM, _NCON), 1)
    cit = jnp.min(jnp.where(d2 == m, it, _NCON), axis=1, keepdims=True)
    row = lax.broadcasted_iota(jnp.int32, (_NSYM, 1), 0)
    pkt_ref[...] = row * 1024 + cit
    d2m_ref[...] = m


def _stage1_body(zr_ref, zi_ref, sym_ref, bn_ref, pkt_ref, d2m_ref,
                 si_ref, ci_ref, sd_ref, cf_ref, dg_ref):
    z = jnp.concatenate([zr_ref[...], zi_ref[...]], axis=1)
    zb = lax.dot_general(z, sym_ref[...], _DN_T,
                         preferred_element_type=jnp.float32)
    rn = jnp.sum(z * z, axis=1, keepdims=True)
    d = (rn + bn_ref[...]) - 2.0 * zb
    mn = jnp.min(d, axis=1, keepdims=True)
    pk = jnp.min(jnp.where(d == mn, pkt_ref[...], _NSYM * 1024),
                 axis=1, keepdims=True)
    dg = jnp.min(jnp.where(d == mn, d2m_ref[...], jnp.inf),
                 axis=1, keepdims=True)
    si_ref[...] = jnp.reshape(lax.shift_right_logical(pk, 10), (1, _RL, 128))
    ci_ref[...] = jnp.reshape(lax.bitwise_and(pk, 1023), (1, _RL, 128))
    sd_ref[...] = jnp.reshape(mn, (1, _RL, 128))
    cf_ref[...] = jnp.reshape(1.0 / (1.0 + mn), (1, _RL, 128))
    dg_ref[...] = jnp.reshape(dg, (1, _RL, 128))


def _onehot_body(si_ref, ci_ref, ohs_ref, ohc_ref):
    it = lax.broadcasted_iota(jnp.int32, (_RO, _NSYM), 1)
    ohs_ref[...] = (it == si_ref[...]).astype(jnp.float32)
    it2 = lax.broadcasted_iota(jnp.int32, (_RO, _NCON), 1)
    ohc_ref[...] = (it2 == ci_ref[...]).astype(jnp.float32)


_stage2_tables = pl.pallas_call(
    _stage2_tables_body,
    out_shape=[
        jax.ShapeDtypeStruct((_NSYM, 1), jnp.int32),
        jax.ShapeDtypeStruct((_NSYM, 1), jnp.float32),
    ],
)

_stage1 = pl.pallas_call(
    _stage1_body,
    grid=(_N // _R,),
    in_specs=[
        pl.BlockSpec((_R, _DIM), lambda i: (i, 0)),
        pl.BlockSpec((_R, _DIM), lambda i: (i, 0)),
        pl.BlockSpec((_NSYM, _D), lambda i: (0, 0)),
        pl.BlockSpec((1, _NSYM), lambda i: (0, 0)),
        pl.BlockSpec((1, _NSYM), lambda i: (0, 0)),
        pl.BlockSpec((1, _NSYM), lambda i: (0, 0)),
    ],
    out_specs=[
        pl.BlockSpec((1, _RL, 128), lambda i: (i, 0, 0)),
        pl.BlockSpec((1, _RL, 128), lambda i: (i, 0, 0)),
        pl.BlockSpec((1, _RL, 128), lambda i: (i, 0, 0)),
        pl.BlockSpec((1, _RL, 128), lambda i: (i, 0, 0)),
        pl.BlockSpec((1, _RL, 128), lambda i: (i, 0, 0)),
    ],
    out_shape=[
        jax.ShapeDtypeStruct((_N // _R, _RL, 128), jnp.int32),
        jax.ShapeDtypeStruct((_N // _R, _RL, 128), jnp.int32),
        jax.ShapeDtypeStruct((_N // _R, _RL, 128), jnp.float32),
        jax.ShapeDtypeStruct((_N // _R, _RL, 128), jnp.float32),
        jax.ShapeDtypeStruct((_N // _R, _RL, 128), jnp.float32),
    ],
    compiler_params=pltpu.CompilerParams(
        dimension_semantics=("parallel",)),
)

_onehots = pl.pallas_call(
    _onehot_body,
    grid=(_N // _RO,),
    in_specs=[
        pl.BlockSpec((_RO, 1), lambda i: (i, 0)),
        pl.BlockSpec((_RO, 1), lambda i: (i, 0)),
    ],
    out_specs=[
        pl.BlockSpec((_RO, _NSYM), lambda i: (i, 0)),
        pl.BlockSpec((_RO, _NCON), lambda i: (i, 0)),
    ],
    out_shape=[
        jax.ShapeDtypeStruct((_N, _NSYM), jnp.float32),
        jax.ShapeDtypeStruct((_N, _NCON), jnp.float32),
    ],
    compiler_params=pltpu.CompilerParams(
        dimension_semantics=("parallel",)),
)

_NW = 32
_BPW = _N // _NW
_CH = 72
_NBUF = 3


def _sc_gather_body(table_hbm, idx_hbm, zre_hbm, zim_hbm, idx_v, b0, b1, b2,
                    gs0, gs1, gs2, ss0, ss1, ss2):
    bufs = (b0, b1, b2)
    gsems = (gs0, gs1, gs2)
    ssems = (ss0, ss1, ss2)
    wid = lax.axis_index("s") * 2 + lax.axis_index("c")
    base = wid * _BPW
    pltpu.sync_copy(idx_hbm.at[pl.ds(base, _BPW)], idx_v)
    nch = _BPW // _CH
    gcp, scp = {}, {}

    def start_gather(c):
        s = c % _NBUF
        gcp[c] = pltpu.async_copy(
            table_hbm.at[idx_v.at[pl.ds(c * _CH, _CH)]], bufs[s], gsems[s])

    for c in range(min(2, nch)):
        start_gather(c)
    for c in range(nch):
        s = c % _NBUF
        gcp[c].wait()
        dst = pl.ds(base + c * _CH, _CH)
        scp[c] = (
            pltpu.async_copy(bufs[s].at[:, pl.ds(0, _DIM)],
                             zre_hbm.at[dst], ssems[s]),
            pltpu.async_copy(bufs[s].at[:, pl.ds(_DIM, _DIM)],
                             zim_hbm.at[dst], ssems[s]),
        )
        if c + 2 < nch:
            if c >= 1:
                scp[c - 1][0].wait()
                scp[c - 1][1].wait()
            start_gather(c + 2)
    for c in range(max(0, nch - 3), nch):
        scp[c][0].wait()
        scp[c][1].wait()


@functools.cache
def _sc_gather():
    return pl.kernel(
        _sc_gather_body,
        out_type=(
            jax.ShapeDtypeStruct((_N, _DIM), jnp.float32),
            jax.ShapeDtypeStruct((_N, _DIM), jnp.float32),
        ),
        mesh=plsc.VectorSubcoreMesh(core_axis_name="c", subcore_axis_name="s"),
        scratch_types=(
            [pltpu.VMEM((_BPW,), jnp.int32)]
            + [pltpu.VMEM((_CH, _D), jnp.float32)] * _NBUF
            + [pltpu.SemaphoreType.DMA] * (2 * _NBUF)
        ),
    )


def kernel(z_real, z_imag, sym, con):
    zr = z_real.reshape(_N, _DIM)
    zi = z_imag.reshape(_N, _DIM)
    bn = jnp.sum(sym**2, axis=-1).reshape(1, _NSYM)
    cn = jnp.sum(con**2, axis=-1).reshape(1, _NCON)
    pkt, d2m = _stage2_tables(sym, con, cn)
    si2, ci2, sd2, cf2, dg = _stage1(
        zr, zi, sym, bn, pkt.reshape(1, _NSYM), d2m.reshape(1, _NSYM))
    zre, zim = _sc_gather()(sym, si2.reshape(_N))
    ohs, ohc = _onehots(si2.reshape(_N, 1), ci2.reshape(_N, 1))
    out_c = lax.complex(zre, zim).reshape(_B, _T, _DIM)
    ls = _SCALE * jnp.sum(sd2)
    lc = _SCALE * jnp.sum(dg)
    return (out_c,
            ohs.reshape(_B, _T, _NSYM),
            ohc.reshape(_B, _T, _NCON),
            ls, lc,
            si2.reshape(_B, _T),
            ci2.reshape(_B, _T),
            cf2.reshape(_B, _T))
